# baseline (speedup 1.0000x reference)
import math

import jax
import jax.numpy as jnp
from jax.experimental import pallas as pl
from jax.experimental.pallas import tpu as pltpu

E = 8
SIGMA = 25.0
D_MODEL = 768
D_FF = 1536
T_FEAT = 256
N_FREQ = T_FEAT // 2
_LN_SIGMA = math.log(SIGMA)
_LOG1000 = math.log(1000.0)


def _moe_kernel(e_ref, t_ref, x_ref, W1_ref, b1_ref, Wt_ref, W2_ref, b2_ref,
                o_ref):
    b = pl.program_id(0)
    t = t_ref[b]

    idx = jax.lax.broadcasted_iota(jnp.int32, (1, N_FREQ), 1).astype(jnp.float32)
    freqs = jnp.exp(idx * (_LOG1000 / (N_FREQ - 1)))
    ang = t * freqs
    temb = jnp.concatenate([jnp.sin(ang), jnp.cos(ang)], axis=-1)
    tvec = jnp.dot(temb.astype(jnp.bfloat16), Wt_ref[...],
                   preferred_element_type=jnp.float32)

    h = jnp.dot(x_ref[...], W1_ref[...], preferred_element_type=jnp.float32)
    h = h + b1_ref[...] + tvec
    h = jax.nn.gelu(h)
    s = jnp.dot(h.astype(jnp.bfloat16), W2_ref[...],
                preferred_element_type=jnp.float32)

    inv_std = jax.lax.rsqrt(
        (jnp.exp(2.0 * t * _LN_SIGMA) - 1.0) / (2.0 * _LN_SIGMA))
    o_ref[...] = (s + b2_ref[...]) * inv_std


@jax.jit
def kernel(x, t, W1, b1, Wt, W2, b2):
    if x.ndim == 2:
        x = x[None]
    if t.ndim == 0:
        t = t * jnp.ones((x.shape[0],), x.dtype)
    B, N, _ = x.shape
    e = jnp.minimum(jnp.floor(t * E).astype(jnp.int32), E - 1)

    b1_3d = b1.reshape(E, 1, D_FF)
    b2_3d = b2.reshape(E, 1, D_MODEL)
    x_bf = x.astype(jnp.bfloat16)
    W1_bf = W1.astype(jnp.bfloat16)
    Wt_bf = Wt.astype(jnp.bfloat16)
    W2_bf = W2.astype(jnp.bfloat16)

    grid_spec = pltpu.PrefetchScalarGridSpec(
        num_scalar_prefetch=2,
        grid=(B,),
        in_specs=[
            pl.BlockSpec((None, N, D_MODEL), lambda b, e, t: (b, 0, 0)),
            pl.BlockSpec((None, D_MODEL, D_FF), lambda b, e, t: (e[b], 0, 0)),
            pl.BlockSpec((None, 1, D_FF), lambda b, e, t: (e[b], 0, 0)),
            pl.BlockSpec((None, T_FEAT, D_FF), lambda b, e, t: (e[b], 0, 0)),
            pl.BlockSpec((None, D_FF, D_MODEL), lambda b, e, t: (e[b], 0, 0)),
            pl.BlockSpec((None, 1, D_MODEL), lambda b, e, t: (e[b], 0, 0)),
        ],
        out_specs=pl.BlockSpec((None, N, D_MODEL), lambda b, e, t: (b, 0, 0)),
    )

    out = pl.pallas_call(
        _moe_kernel,
        grid_spec=grid_spec,
        out_shape=jax.ShapeDtypeStruct((B, N, D_MODEL), jnp.float32),
    )(e, t, x_bf, W1_bf, b1_3d, Wt_bf, W2_bf, b2_3d)
    return out


# speedup vs baseline: 2.0932x; 2.0932x over previous
import math

import jax
import jax.numpy as jnp
from jax.experimental import pallas as pl
from jax.experimental.pallas import tpu as pltpu

E = 8
SIGMA = 25.0
D_MODEL = 768
D_FF = 1536
T_FEAT = 256
N_FREQ = T_FEAT // 2
_LN_SIGMA = math.log(SIGMA)
_LOG1000 = math.log(1000.0)


def _moe_kernel(order_ref, e_ref, t_ref, x_ref, W1_ref, b1_ref, Wt_ref,
                W2_ref, b2_ref, o_ref):
    i = pl.program_id(0)
    t = t_ref[order_ref[i]]

    idx = jax.lax.broadcasted_iota(jnp.int32, (1, N_FREQ), 1).astype(jnp.float32)
    freqs = jnp.exp(idx * (_LOG1000 / (N_FREQ - 1)))
    ang = t * freqs
    temb = jnp.concatenate([jnp.sin(ang), jnp.cos(ang)], axis=-1)
    tvec = jnp.dot(temb, Wt_ref[...], preferred_element_type=jnp.float32)

    h = jnp.dot(x_ref[...], W1_ref[...], preferred_element_type=jnp.float32)
    h = h + b1_ref[...] + tvec
    h = jax.nn.gelu(h)
    s = jnp.dot(h, W2_ref[...], preferred_element_type=jnp.float32)

    inv_std = jax.lax.rsqrt(
        (jnp.exp(2.0 * t * _LN_SIGMA) - 1.0) / (2.0 * _LN_SIGMA))
    o_ref[...] = (s + b2_ref[...]) * inv_std


@jax.jit
def kernel(x, t, W1, b1, Wt, W2, b2):
    if x.ndim == 2:
        x = x[None]
    if t.ndim == 0:
        t = t * jnp.ones((x.shape[0],), x.dtype)
    B, N, _ = x.shape
    e = jnp.minimum(jnp.floor(t * E).astype(jnp.int32), E - 1)
    order = jnp.argsort(e).astype(jnp.int32)
    e_s = e[order]

    b1_3d = b1.reshape(E, 1, D_FF)
    b2_3d = b2.reshape(E, 1, D_MODEL)

    grid_spec = pltpu.PrefetchScalarGridSpec(
        num_scalar_prefetch=3,
        grid=(B,),
        in_specs=[
            pl.BlockSpec((None, N, D_MODEL), lambda i, p, e, t: (p[i], 0, 0)),
            pl.BlockSpec((None, D_MODEL, D_FF), lambda i, p, e, t: (e[i], 0, 0)),
            pl.BlockSpec((None, 1, D_FF), lambda i, p, e, t: (e[i], 0, 0)),
            pl.BlockSpec((None, T_FEAT, D_FF), lambda i, p, e, t: (e[i], 0, 0)),
            pl.BlockSpec((None, D_FF, D_MODEL), lambda i, p, e, t: (e[i], 0, 0)),
            pl.BlockSpec((None, 1, D_MODEL), lambda i, p, e, t: (e[i], 0, 0)),
        ],
        out_specs=pl.BlockSpec((None, N, D_MODEL), lambda i, p, e, t: (p[i], 0, 0)),
    )

    out = pl.pallas_call(
        _moe_kernel,
        grid_spec=grid_spec,
        out_shape=jax.ShapeDtypeStruct((B, N, D_MODEL), jnp.float32),
        compiler_params=pltpu.CompilerParams(
            dimension_semantics=("arbitrary",)),
    )(order, e_s, t, x, W1, b1_3d, Wt, W2, b2_3d)
    return out


# speedup vs baseline: 2.0943x; 1.0005x over previous
import math

import jax
import jax.numpy as jnp
from jax.experimental import pallas as pl
from jax.experimental.pallas import tpu as pltpu

E = 8
SIGMA = 25.0
D_MODEL = 768
D_FF = 1536
T_FEAT = 256
N_FREQ = T_FEAT // 2
_LN_SIGMA = math.log(SIGMA)
_LOG1000 = math.log(1000.0)


def _moe_kernel(order_ref, e_ref, t_ref, x_ref, W1_ref, b1_ref, Wt_ref,
                W2_ref, b2_ref, o_ref):
    i = pl.program_id(0)
    t = t_ref[order_ref[i]]

    idx = jax.lax.broadcasted_iota(jnp.int32, (1, N_FREQ), 1).astype(jnp.float32)
    freqs = jnp.exp(idx * (_LOG1000 / (N_FREQ - 1)))
    ang = t * freqs
    temb = jnp.concatenate([jnp.sin(ang), jnp.cos(ang)], axis=-1)
    tvec = jnp.dot(temb, Wt_ref[...], preferred_element_type=jnp.float32)

    h = jnp.dot(x_ref[...], W1_ref[...], preferred_element_type=jnp.float32)
    h = h + b1_ref[...] + tvec
    h = jax.nn.gelu(h)
    s = jnp.dot(h, W2_ref[...], preferred_element_type=jnp.float32)

    inv_std = jax.lax.rsqrt(
        (jnp.exp(2.0 * t * _LN_SIGMA) - 1.0) / (2.0 * _LN_SIGMA))
    o_ref[...] = (s + b2_ref[...]) * inv_std


@jax.jit
def kernel(x, t, W1, b1, Wt, W2, b2):
    if x.ndim == 2:
        x = x[None]
    if t.ndim == 0:
        t = t * jnp.ones((x.shape[0],), x.dtype)
    B, N, _ = x.shape
    e = jnp.minimum(jnp.floor(t * E).astype(jnp.int32), E - 1)
    order = jnp.argsort(e).astype(jnp.int32)
    e_s = e[order]

    b1_3d = b1.reshape(E, 1, D_FF)
    b2_3d = b2.reshape(E, 1, D_MODEL)

    grid_spec = pltpu.PrefetchScalarGridSpec(
        num_scalar_prefetch=3,
        grid=(B,),
        in_specs=[
            pl.BlockSpec((None, N, D_MODEL), lambda i, p, e, t: (p[i], 0, 0)),
            pl.BlockSpec((None, D_MODEL, D_FF), lambda i, p, e, t: (e[i], 0, 0)),
            pl.BlockSpec((None, 1, D_FF), lambda i, p, e, t: (e[i], 0, 0)),
            pl.BlockSpec((None, T_FEAT, D_FF), lambda i, p, e, t: (e[i], 0, 0)),
            pl.BlockSpec((None, D_FF, D_MODEL), lambda i, p, e, t: (e[i], 0, 0)),
            pl.BlockSpec((None, 1, D_MODEL), lambda i, p, e, t: (e[i], 0, 0)),
        ],
        out_specs=pl.BlockSpec((None, N, D_MODEL), lambda i, p, e, t: (p[i], 0, 0)),
    )

    out = pl.pallas_call(
        _moe_kernel,
        grid_spec=grid_spec,
        out_shape=jax.ShapeDtypeStruct((B, N, D_MODEL), jnp.float32),
        compiler_params=pltpu.CompilerParams(
            dimension_semantics=("parallel",)),
    )(order, e_s, t, x, W1, b1_3d, Wt, W2, b2_3d)
    return out
